# Initial kernel scaffold; baseline (speedup 1.0000x reference)
import jax
import jax.numpy as jnp
from jax.experimental import pallas as pl


def kernel(x, emb_table, pos_table):
    raise NotImplementedError("write your pallas kernel here")


# speedup vs baseline: 2.0360x; 2.0360x over previous
import functools

import jax
import jax.numpy as jnp
from jax import lax
from jax.experimental import pallas as pl
from jax.experimental.pallas import tpu as pltpu
from jax.experimental.pallas import tpu_sc as plsc

DM = 64
SEQ = 200
BATCH = 4096

NC, NS = 2, 16
NW = NC * NS
ROWS = BATCH * SEQ
RPW = ROWS // NW
CHUNK = 128
NCH = RPW // CHUNK
LANES = 16
GRP = DM // LANES


def _make_kernel():
    mesh = plsc.VectorSubcoreMesh(core_axis_name="c", subcore_axis_name="s")

    @functools.partial(
        pl.kernel,
        mesh=mesh,
        out_type=jax.ShapeDtypeStruct((ROWS, DM), jnp.float32),
        compiler_params=pltpu.CompilerParams(use_tc_tiling_on_sc=False),
        scratch_types=[
            pltpu.VMEM((RPW,), jnp.int32),
            pltpu.VMEM((SEQ, DM), jnp.float32),
            pltpu.VMEM((CHUNK, DM), jnp.float32),
            pltpu.SemaphoreType.DMA,
        ],
    )
    def emb_kernel(idx_hbm, table_hbm, pos_hbm, out_hbm, idx_v, pos_v, rows_v, sem):
        wid = lax.axis_index("s") * NC + lax.axis_index("c")
        base = wid * RPW
        pltpu.sync_copy(idx_hbm.at[pl.ds(base, RPW)], idx_v)
        pltpu.sync_copy(pos_hbm, pos_v)

        def chunk_body(j, carry):
            pltpu.async_copy(
                table_hbm.at[idx_v.at[pl.ds(j * CHUNK, CHUNK)]], rows_v, sem
            ).wait()

            def row_body(r, carry2):
                s = lax.rem(j * CHUNK + r, SEQ)
                for g in range(GRP):
                    pg = pos_v[s, pl.ds(g * LANES, LANES)]
                    plsc.addupdate(rows_v.at[r, pl.ds(g * LANES, LANES)], pg)
                return carry2

            lax.fori_loop(0, CHUNK, row_body, 0)
            pltpu.sync_copy(rows_v, out_hbm.at[pl.ds(base + j * CHUNK, CHUNK)])
            return carry

        lax.fori_loop(0, NCH, chunk_body, 0)

    return emb_kernel


_emb = _make_kernel()


@jax.jit
def kernel(x, emb_table, pos_table):
    flat = x.reshape(-1)
    out = _emb(flat, emb_table, pos_table)
    return out.reshape(BATCH, SEQ, DM)


# speedup vs baseline: 2.8494x; 1.3995x over previous
import functools

import jax
import jax.numpy as jnp
from jax import lax
from jax.experimental import pallas as pl
from jax.experimental.pallas import tpu as pltpu
from jax.experimental.pallas import tpu_sc as plsc

DM = 64
SEQ = 200
BATCH = 4096

NC, NS = 2, 16
NW = NC * NS
ROWS = BATCH * SEQ
RPW = ROWS // NW
CHUNK = 128
NCH = RPW // CHUNK
NBUF = 4
LANES = 16
GRP = DM // LANES


def _make_kernel():
    mesh = plsc.VectorSubcoreMesh(core_axis_name="c", subcore_axis_name="s")

    @functools.partial(
        pl.kernel,
        mesh=mesh,
        out_type=jax.ShapeDtypeStruct((ROWS, DM), jnp.float32),
        compiler_params=pltpu.CompilerParams(use_tc_tiling_on_sc=False),
        scratch_types=[
            pltpu.VMEM((RPW,), jnp.int32),
            pltpu.VMEM((SEQ, DM), jnp.float32),
            pltpu.VMEM((NBUF, CHUNK, DM), jnp.float32),
            pltpu.SemaphoreType.DMA((NBUF,)),
            pltpu.SemaphoreType.DMA((NBUF,)),
        ],
    )
    def emb_kernel(idx_hbm, table_hbm, pos_hbm, out_hbm,
                   idx_v, pos_v, rows_v, gsem, osem):
        wid = lax.axis_index("s") * NC + lax.axis_index("c")
        base = wid * RPW
        pltpu.sync_copy(idx_hbm.at[pl.ds(base, RPW)], idx_v)
        pltpu.sync_copy(pos_hbm, pos_v)

        def gather(j, b):
            return pltpu.make_async_copy(
                table_hbm.at[idx_v.at[pl.ds(j * CHUNK, CHUNK)]],
                rows_v.at[b], gsem.at[b])

        def outcopy(j, b):
            return pltpu.make_async_copy(
                rows_v.at[b], out_hbm.at[pl.ds(base + j * CHUNK, CHUNK)],
                osem.at[b])

        def add_pos(j, b):
            s0 = lax.rem(j * CHUNK, SEQ)

            @plsc.parallel_loop(0, CHUNK, 1, unroll=4)
            def _(r):
                s = lax.rem(s0 + r, SEQ)
                for g in range(GRP):
                    plsc.addupdate(rows_v.at[b, r, pl.ds(g * LANES, LANES)],
                                   pos_v[s, pl.ds(g * LANES, LANES)])

        for b in range(NBUF):
            gather(b, b).start()

        def outer(t, carry):
            jj = t * NBUF
            for b in range(NBUF):
                j = jj + b
                gather(j, b).wait()
                add_pos(j, b)
                outcopy(j, b).start()
                jp = j - 1
                bp = (b - 1) % NBUF
                jn = jp + NBUF

                @pl.when((jp >= 0) & (jn < NCH))
                def _():
                    outcopy(jp, bp).wait()
                    gather(jn, bp).start()

            return carry

        lax.fori_loop(0, NCH // NBUF, outer, 0)
        for k in range(NBUF):
            j = NCH - NBUF + k
            outcopy(j, j % NBUF).wait()

    return emb_kernel


_emb = _make_kernel()


@jax.jit
def kernel(x, emb_table, pos_table):
    flat = x.reshape(-1)
    out = _emb(flat, emb_table, pos_table)
    return out.reshape(BATCH, SEQ, DM)


# speedup vs baseline: 2.8622x; 1.0045x over previous
import functools

import jax
import jax.numpy as jnp
from jax import lax
from jax.experimental import pallas as pl
from jax.experimental.pallas import tpu as pltpu
from jax.experimental.pallas import tpu_sc as plsc

DM = 64
SEQ = 200
BATCH = 4096

NC, NS = 2, 16
NW = NC * NS
ROWS = BATCH * SEQ
RPW = ROWS // NW
BPW = BATCH // NW
G1 = 128
G2 = SEQ - G1
NBUF = 4
LANES = 16
GRP = DM // LANES


def _make_kernel():
    mesh = plsc.VectorSubcoreMesh(core_axis_name="c", subcore_axis_name="s")

    @functools.partial(
        pl.kernel,
        mesh=mesh,
        out_type=jax.ShapeDtypeStruct((BATCH, SEQ, DM), jnp.float32),
        compiler_params=pltpu.CompilerParams(use_tc_tiling_on_sc=False),
        scratch_types=[
            pltpu.VMEM((RPW,), jnp.int32),
            pltpu.VMEM((SEQ, DM), jnp.float32),
            pltpu.VMEM((NBUF, SEQ, DM), jnp.float32),
            pltpu.SemaphoreType.DMA((NBUF,)),
            pltpu.SemaphoreType.DMA((NBUF,)),
        ],
    )
    def emb_kernel(idx_hbm, table_hbm, pos_hbm, out_hbm,
                   idx_v, pos_v, rows_v, gsem, osem):
        wid = lax.axis_index("s") * NC + lax.axis_index("c")
        base = wid * RPW
        bbase = wid * BPW
        pltpu.sync_copy(idx_hbm.at[pl.ds(base, RPW)], idx_v)
        pltpu.sync_copy(pos_hbm, pos_v)

        def gather1(j, b):
            return pltpu.make_async_copy(
                table_hbm.at[idx_v.at[pl.ds(j * SEQ, G1)]],
                rows_v.at[b, pl.ds(0, G1)], gsem.at[b])

        def gather2(j, b):
            return pltpu.make_async_copy(
                table_hbm.at[idx_v.at[pl.ds(j * SEQ + G1, G2)]],
                rows_v.at[b, pl.ds(G1, G2)], gsem.at[b])

        def outcopy(j, b):
            return pltpu.make_async_copy(
                rows_v.at[b], out_hbm.at[bbase + j], osem.at[b])

        def add_pos(b):
            @plsc.parallel_loop(0, SEQ, 1, unroll=4)
            def _(r):
                for g in range(GRP):
                    plsc.addupdate(rows_v.at[b, r, pl.ds(g * LANES, LANES)],
                                   pos_v[r, pl.ds(g * LANES, LANES)])

        for b in range(NBUF):
            gather1(b, b).start()
            gather2(b, b).start()

        def outer(t, carry):
            jj = t * NBUF
            for b in range(NBUF):
                j = jj + b
                gather1(j, b).wait()
                gather2(j, b).wait()
                add_pos(b)
                outcopy(j, b).start()
                jp = j - 1
                bp = (b - 1) % NBUF
                jn = jp + NBUF

                @pl.when((jp >= 0) & (jn < BPW))
                def _():
                    outcopy(jp, bp).wait()
                    gather1(jn, bp).start()
                    gather2(jn, bp).start()

            return carry

        lax.fori_loop(0, BPW // NBUF, outer, 0)
        for k in range(NBUF):
            j = BPW - NBUF + k
            outcopy(j, j % NBUF).wait()

    return emb_kernel


_emb = _make_kernel()


@jax.jit
def kernel(x, emb_table, pos_table):
    flat = x.reshape(-1)
    return _emb(flat, emb_table, pos_table)
